# baseline (speedup 1.0000x reference)
import jax
import jax.numpy as jnp
from jax.experimental import pallas as pl
from jax.experimental.pallas import tpu as pltpu

NODE_NUM = 100
HIDDEN_SIZE = 4096
NCHUNK = 2
CHUNK_COLS = HIDDEN_SIZE // NCHUNK


def _dma_body(in_hbm, out_hbm, buf, insem, outsem):
    for c in range(NCHUNK):
        pltpu.make_async_copy(
            in_hbm.at[:, pl.ds(c * CHUNK_COLS, CHUNK_COLS)],
            buf.at[:, pl.ds(c * CHUNK_COLS, CHUNK_COLS)],
            insem.at[c],
        ).start()
    for c in range(NCHUNK):
        pltpu.make_async_copy(
            in_hbm.at[:, pl.ds(c * CHUNK_COLS, CHUNK_COLS)],
            buf.at[:, pl.ds(c * CHUNK_COLS, CHUNK_COLS)],
            insem.at[c],
        ).wait()
        pltpu.make_async_copy(
            buf.at[:, pl.ds(c * CHUNK_COLS, CHUNK_COLS)],
            out_hbm.at[:, pl.ds(c * CHUNK_COLS, CHUNK_COLS)],
            outsem.at[c],
        ).start()
    for c in range(NCHUNK):
        pltpu.make_async_copy(
            buf.at[:, pl.ds(c * CHUNK_COLS, CHUNK_COLS)],
            out_hbm.at[:, pl.ds(c * CHUNK_COLS, CHUNK_COLS)],
            outsem.at[c],
        ).wait()


def kernel(node_table):
    return pl.pallas_call(
        _dma_body,
        out_shape=jax.ShapeDtypeStruct((NODE_NUM, HIDDEN_SIZE), jnp.float32),
        in_specs=[pl.BlockSpec(memory_space=pl.ANY)],
        out_specs=pl.BlockSpec(memory_space=pl.ANY),
        scratch_shapes=[
            pltpu.VMEM((NODE_NUM, HIDDEN_SIZE), jnp.float32),
            pltpu.SemaphoreType.DMA((NCHUNK,)),
            pltpu.SemaphoreType.DMA((NCHUNK,)),
        ],
    )(node_table)


# speedup vs baseline: 1.0119x; 1.0119x over previous
import jax
import jax.numpy as jnp
from jax.experimental import pallas as pl
from jax.experimental.pallas import tpu as pltpu

NODE_NUM = 100
HIDDEN_SIZE = 4096
ROW_CHUNKS = ((0, 32), (32, 32), (64, 32), (96, 4))
NCHUNK = len(ROW_CHUNKS)


def _dma_body(in_hbm, out_hbm, buf, insem, outsem):
    for c, (off, sz) in enumerate(ROW_CHUNKS):
        pltpu.make_async_copy(
            in_hbm.at[pl.ds(off, sz), :],
            buf.at[pl.ds(off, sz), :],
            insem.at[c],
        ).start()
    for c, (off, sz) in enumerate(ROW_CHUNKS):
        pltpu.make_async_copy(
            in_hbm.at[pl.ds(off, sz), :],
            buf.at[pl.ds(off, sz), :],
            insem.at[c],
        ).wait()
        pltpu.make_async_copy(
            buf.at[pl.ds(off, sz), :],
            out_hbm.at[pl.ds(off, sz), :],
            outsem.at[c],
        ).start()
    for c, (off, sz) in enumerate(ROW_CHUNKS):
        pltpu.make_async_copy(
            buf.at[pl.ds(off, sz), :],
            out_hbm.at[pl.ds(off, sz), :],
            outsem.at[c],
        ).wait()


def kernel(node_table):
    return pl.pallas_call(
        _dma_body,
        out_shape=jax.ShapeDtypeStruct((NODE_NUM, HIDDEN_SIZE), jnp.float32),
        in_specs=[pl.BlockSpec(memory_space=pl.ANY)],
        out_specs=pl.BlockSpec(memory_space=pl.ANY),
        scratch_shapes=[
            pltpu.VMEM((NODE_NUM, HIDDEN_SIZE), jnp.float32),
            pltpu.SemaphoreType.DMA((NCHUNK,)),
            pltpu.SemaphoreType.DMA((NCHUNK,)),
        ],
    )(node_table)
